# baseline (speedup 1.0000x reference)
import functools
import math

import jax
import jax.numpy as jnp
from jax import lax
from jax.experimental import pallas as pl

B, CIN, H, W = 8, 768, 64, 64
COUT, K = 32, 8192
HP, WP = H // 2, W // 2
NTOK_B = HP * WP
NTOK = B * NTOK_B
K_TILE = 2048


def _vq_kernel(inp_ref, cbt_ref, x2_ref, e2_ref, g_ref, b_ref, out_ref):
    xqb_bf = inp_ref[...]
    cbt = cbt_ref[...]
    cbt_bf = cbt.astype(jnp.bfloat16)
    x2 = x2_ref[...]

    run_min = jnp.full((1, NTOK_B), jnp.inf, jnp.float32)
    run_idx = jnp.zeros((1, NTOK_B), jnp.int32)
    for t in range(K // K_TILE):
        cbt_tile = cbt_bf[:, t * K_TILE:(t + 1) * K_TILE]
        e2 = e2_ref[t * K_TILE:(t + 1) * K_TILE, :]
        mm = lax.dot_general(cbt_tile, xqb_bf, (((0,), (0,)), ((), ())),
                             preferred_element_type=jnp.float32)
        d2 = (x2 - 2.0 * mm) + e2
        m = jnp.min(d2, axis=0, keepdims=True)
        ids = lax.broadcasted_iota(jnp.int32, d2.shape, 0) + t * K_TILE
        cand = jnp.min(jnp.where(d2 == m, ids, K), axis=0, keepdims=True)
        upd = m < run_min
        run_idx = jnp.where(upd, cand, run_idx)
        run_min = jnp.where(upd, m.astype(jnp.bfloat16).astype(jnp.float32),
                            run_min)

    quant = jnp.zeros((COUT, NTOK_B), jnp.float32)
    for t in range(K // K_TILE):
        cbf_tile = cbt[:, t * K_TILE:(t + 1) * K_TILE]
        ids = (lax.broadcasted_iota(jnp.int32, (K_TILE, NTOK_B), 0) + t * K_TILE)
        onehot = (ids == run_idx).astype(jnp.float32)
        quant = quant + lax.dot_general(
            cbf_tile, onehot, (((1,), (0,)), ((), ())),
            precision=lax.Precision.HIGHEST,
            preferred_element_type=jnp.float32)

    l = lax.broadcasted_iota(jnp.int32, (1, NTOK_B), 1)
    hh = l // WP
    ww = l % WP
    ye = (hh + 1).astype(jnp.float32) / (HP + 1e-6) * (2.0 * math.pi)
    xe = (ww + 1).astype(jnp.float32) / (WP + 1e-6) * (2.0 * math.pi)
    c = lax.broadcasted_iota(jnp.int32, (COUT, 1), 0)
    cm = c % (COUT // 2)
    expo = 2.0 * jnp.floor(cm.astype(jnp.float32) / 2.0) / (COUT // 2)
    dim_t = jnp.exp(expo * math.log(10000.0))
    ang = jnp.where(c < COUT // 2, ye, xe) / dim_t
    pos = jnp.where(cm % 2 == 0, jnp.sin(ang), jnp.cos(ang))
    xq2 = quant + pos

    mu = jnp.mean(xq2, axis=0, keepdims=True)
    var = jnp.mean((xq2 - mu) ** 2, axis=0, keepdims=True)
    out_ref[...] = (xq2 - mu) / jnp.sqrt(var + 1e-5) * g_ref[...] + b_ref[...]


def kernel(img, conv_w, conv_b, codebook, ln_g, ln_b):
    b, cin, Hh, Ww = img.shape
    cout = conv_w.shape[0]
    x = img.reshape(b, cin, Hh // 2, 2, Ww // 2, 2).max(axis=(3, 5))
    xq = jnp.einsum('bchw,oc->bohw', x, conv_w) + conv_b[None, :, None, None]
    inputs = jnp.transpose(xq, (0, 2, 3, 1)).reshape(NTOK, cout)
    inputs_bf_t = inputs.astype(jnp.bfloat16).T
    x2 = jnp.sum(inputs * inputs, axis=1, keepdims=True)
    e2 = jnp.sum(codebook * codebook, axis=1)[:, None]
    cbt = codebook.T

    out = pl.pallas_call(
        _vq_kernel,
        grid=(b,),
        in_specs=[
            pl.BlockSpec((COUT, NTOK_B), lambda i: (0, i)),
            pl.BlockSpec((COUT, K), lambda i: (0, 0)),
            pl.BlockSpec((1, NTOK_B), lambda i: (0, i)),
            pl.BlockSpec((K, 1), lambda i: (0, 0)),
            pl.BlockSpec((COUT, 1), lambda i: (0, 0)),
            pl.BlockSpec((COUT, 1), lambda i: (0, 0)),
        ],
        out_specs=pl.BlockSpec((COUT, NTOK_B), lambda i: (0, i)),
        out_shape=jax.ShapeDtypeStruct((COUT, NTOK), jnp.float32),
    )(inputs_bf_t, cbt, x2.reshape(1, NTOK), e2,
      ln_g.reshape(cout, 1), ln_b.reshape(cout, 1))

    xqn = out.T.reshape(b, HP * WP, cout)
    visual_mask = jnp.ones((b, HP * WP), dtype=jnp.int32)
    return xqn, visual_mask


# speedup vs baseline: 1.1942x; 1.1942x over previous
import functools
import math

import jax
import jax.numpy as jnp
from jax import lax
from jax.experimental import pallas as pl

B, CIN, H, W = 8, 768, 64, 64
COUT, K = 32, 8192
HP, WP = H // 2, W // 2
NTOK_B = HP * WP
NTOK = B * NTOK_B
K_TILE = 2048


def _vq_kernel(inp_ref, cbt_ref, x2_ref, e2_ref, g_ref, b_ref, out_ref):
    xqb_bf = inp_ref[...]
    cbt = cbt_ref[...]
    cbt_bf = cbt.astype(jnp.bfloat16)
    x2 = x2_ref[...]

    run_min = jnp.full((1, NTOK_B), jnp.inf, jnp.float32)
    run_idx = jnp.zeros((1, NTOK_B), jnp.int32)
    for t in range(K // K_TILE):
        cbt_tile = cbt_bf[:, t * K_TILE:(t + 1) * K_TILE]
        e2 = e2_ref[t * K_TILE:(t + 1) * K_TILE, :]
        mm = lax.dot_general(cbt_tile, xqb_bf, (((0,), (0,)), ((), ())),
                             preferred_element_type=jnp.float32)
        d2 = (x2 - 2.0 * mm) + e2
        m = jnp.min(d2, axis=0, keepdims=True)
        ids = lax.broadcasted_iota(jnp.int32, d2.shape, 0) + t * K_TILE
        cand = jnp.min(jnp.where(d2 == m, ids, K), axis=0, keepdims=True)
        upd = m < run_min
        run_idx = jnp.where(upd, cand, run_idx)
        run_min = jnp.where(upd, m.astype(jnp.bfloat16).astype(jnp.float32),
                            run_min)

    cb_hi = cbt.astype(jnp.bfloat16)
    r1 = cbt - cb_hi.astype(jnp.float32)
    cb_mid = r1.astype(jnp.bfloat16)
    cb_lo = (r1 - cb_mid.astype(jnp.float32)).astype(jnp.bfloat16)
    quant = jnp.zeros((COUT, NTOK_B), jnp.float32)
    for t in range(K // K_TILE):
        sl = slice(t * K_TILE, (t + 1) * K_TILE)
        ids = (lax.broadcasted_iota(jnp.int32, (K_TILE, NTOK_B), 0) + t * K_TILE)
        onehot = (ids == run_idx).astype(jnp.bfloat16)
        for part in (cb_hi, cb_mid, cb_lo):
            quant = quant + lax.dot_general(
                part[:, sl], onehot, (((1,), (0,)), ((), ())),
                preferred_element_type=jnp.float32)

    l = lax.broadcasted_iota(jnp.int32, (1, NTOK_B), 1)
    hh = l // WP
    ww = l % WP
    ye = (hh + 1).astype(jnp.float32) / (HP + 1e-6) * (2.0 * math.pi)
    xe = (ww + 1).astype(jnp.float32) / (WP + 1e-6) * (2.0 * math.pi)
    c = lax.broadcasted_iota(jnp.int32, (COUT, 1), 0)
    cm = c % (COUT // 2)
    expo = 2.0 * jnp.floor(cm.astype(jnp.float32) / 2.0) / (COUT // 2)
    dim_t = jnp.exp(expo * math.log(10000.0))
    ang = jnp.where(c < COUT // 2, ye, xe) / dim_t
    pos = jnp.where(cm % 2 == 0, jnp.sin(ang), jnp.cos(ang))
    xq2 = quant + pos

    mu = jnp.mean(xq2, axis=0, keepdims=True)
    var = jnp.mean((xq2 - mu) ** 2, axis=0, keepdims=True)
    out_ref[...] = (xq2 - mu) / jnp.sqrt(var + 1e-5) * g_ref[...] + b_ref[...]


def kernel(img, conv_w, conv_b, codebook, ln_g, ln_b):
    b, cin, Hh, Ww = img.shape
    cout = conv_w.shape[0]
    x = img.reshape(b, cin, Hh // 2, 2, Ww // 2, 2).max(axis=(3, 5))
    xq = jnp.einsum('bchw,oc->bohw', x, conv_w) + conv_b[None, :, None, None]
    inputs = jnp.transpose(xq, (0, 2, 3, 1)).reshape(NTOK, cout)
    inputs_bf_t = inputs.astype(jnp.bfloat16).T
    x2 = jnp.sum(inputs * inputs, axis=1, keepdims=True)
    e2 = jnp.sum(codebook * codebook, axis=1)[:, None]
    cbt = codebook.T

    out = pl.pallas_call(
        _vq_kernel,
        grid=(b,),
        in_specs=[
            pl.BlockSpec((COUT, NTOK_B), lambda i: (0, i)),
            pl.BlockSpec((COUT, K), lambda i: (0, 0)),
            pl.BlockSpec((1, NTOK_B), lambda i: (0, i)),
            pl.BlockSpec((K, 1), lambda i: (0, 0)),
            pl.BlockSpec((COUT, 1), lambda i: (0, 0)),
            pl.BlockSpec((COUT, 1), lambda i: (0, 0)),
        ],
        out_specs=pl.BlockSpec((COUT, NTOK_B), lambda i: (0, i)),
        out_shape=jax.ShapeDtypeStruct((COUT, NTOK), jnp.float32),
    )(inputs_bf_t, cbt, x2.reshape(1, NTOK), e2,
      ln_g.reshape(cout, 1), ln_b.reshape(cout, 1))

    xqn = out.T.reshape(b, HP * WP, cout)
    visual_mask = jnp.ones((b, HP * WP), dtype=jnp.int32)
    return xqn, visual_mask
